# Initial kernel scaffold; baseline (speedup 1.0000x reference)
import jax
import jax.numpy as jnp
from jax.experimental import pallas as pl


def kernel(x, edge_index, W1, b1, W2, b2):
    raise NotImplementedError("write your pallas kernel here")


# speedup vs baseline: 21.0341x; 21.0341x over previous
import functools

import jax
import jax.numpy as jnp
from jax import lax
from jax.experimental import pallas as pl
from jax.experimental.pallas import tpu as pltpu
from jax.experimental.pallas import tpu_sc as plsc

N = 10000
E = 320000
D_IN = 128
H1 = 50
H1P = 64
H2 = 16

NC = 2
NS = 16
NW = NC * NS
EPW = E // NW
CHUNK = 80
NCHUNK = EPW // CHUNK
NP = 10240
RPW = NP // NS

_mesh = plsc.VectorSubcoreMesh(core_axis_name="c", subcore_axis_name="s")


def _make_agg_kernel(D):
    @functools.partial(
        pl.kernel,
        mesh=_mesh,
        out_type=jax.ShapeDtypeStruct((NC, NP, D), jnp.float32),
        scratch_types=[
            pltpu.VMEM((NCHUNK, CHUNK), jnp.int32),
            pltpu.VMEM((NCHUNK, CHUNK), jnp.int32),
            pltpu.VMEM((CHUNK, D), jnp.float32),
            pltpu.VMEM_SHARED((NP, D), jnp.float32),
            pltpu.SemaphoreType.DMA,
        ],
        compiler_params=pltpu.CompilerParams(use_tc_tiling_on_sc=False),
    )
    def agg(hs_hbm, src_hbm, dst_hbm, zeros_hbm, out_hbm,
            sidx_v, didx_v, rows_v, agg_sp, sem):
        c = lax.axis_index("c")
        s = lax.axis_index("s")
        wid = c * NS + s

        pltpu.sync_copy(zeros_hbm.at[pl.ds(s * RPW, RPW)],
                        agg_sp.at[pl.ds(s * RPW, RPW)])
        pltpu.sync_copy(src_hbm.at[wid], sidx_v)
        pltpu.sync_copy(dst_hbm.at[wid], didx_v)
        plsc.subcore_barrier()

        def body(k, _):
            pltpu.async_copy(hs_hbm.at[sidx_v.at[k]], rows_v, sem).wait()
            pltpu.sync_copy(rows_v, agg_sp.at[didx_v.at[k]], add=True)
            return ()

        lax.fori_loop(0, NCHUNK, body, ())

        plsc.subcore_barrier()
        pltpu.sync_copy(agg_sp.at[pl.ds(s * RPW, RPW)],
                        out_hbm.at[c, pl.ds(s * RPW, RPW)])

    return agg


_agg64 = _make_agg_kernel(H1P)
_agg16 = _make_agg_kernel(H2)


_RB = 1000


def _tc_a_body(x_ref, w_ref, degp_ref, hs_ref, dinv_ref):
    deg = degp_ref[0, :, 0] + degp_ref[1, :, 0] + 1.0
    dinv = lax.rsqrt(deg)
    dinv_ref[...] = dinv[:, None]
    hs_ref[...] = jnp.dot(x_ref[...], w_ref[...],
                          preferred_element_type=jnp.float32) * dinv[:, None]


_tc_a = pl.pallas_call(
    _tc_a_body,
    grid=(N // _RB,),
    in_specs=[
        pl.BlockSpec((_RB, D_IN), lambda i: (i, 0)),
        pl.BlockSpec((D_IN, H1P), lambda i: (0, 0)),
        pl.BlockSpec((NC, _RB, H2), lambda i: (0, i, 0)),
    ],
    out_specs=[
        pl.BlockSpec((_RB, H1P), lambda i: (i, 0)),
        pl.BlockSpec((_RB, 1), lambda i: (i, 0)),
    ],
    out_shape=[
        jax.ShapeDtypeStruct((N, H1P), jnp.float32),
        jax.ShapeDtypeStruct((N, 1), jnp.float32),
    ],
)


def _tc_b_body(aggp_ref, hs1_ref, dinv_ref, w_ref, b_ref, hs2_ref):
    agg = aggp_ref[0] + aggp_ref[1] + hs1_ref[...]
    dinv = dinv_ref[...]
    h1 = jnp.maximum(agg * dinv + b_ref[...][None, :], 0.0)
    hs2_ref[...] = jnp.dot(h1, w_ref[...],
                           preferred_element_type=jnp.float32) * dinv


_tc_b = pl.pallas_call(
    _tc_b_body,
    grid=(N // _RB,),
    in_specs=[
        pl.BlockSpec((NC, _RB, H1P), lambda i: (0, i, 0)),
        pl.BlockSpec((_RB, H1P), lambda i: (i, 0)),
        pl.BlockSpec((_RB, 1), lambda i: (i, 0)),
        pl.BlockSpec((H1P, H2), lambda i: (0, 0)),
        pl.BlockSpec((H1P,), lambda i: (0,)),
    ],
    out_specs=pl.BlockSpec((_RB, H2), lambda i: (i, 0)),
    out_shape=jax.ShapeDtypeStruct((N, H2), jnp.float32),
)


def _tc_c_body(aggp_ref, hs2_ref, dinv_ref, b_ref, out_ref):
    agg = aggp_ref[0] + aggp_ref[1] + hs2_ref[...]
    h = agg * dinv_ref[...] + b_ref[...][None, :]
    m = jnp.max(h, axis=1, keepdims=True)
    lse = jnp.log(jnp.sum(jnp.exp(h - m), axis=1, keepdims=True))
    out_ref[...] = h - m - lse


_tc_c = pl.pallas_call(
    _tc_c_body,
    grid=(N // _RB,),
    in_specs=[
        pl.BlockSpec((NC, _RB, H2), lambda i: (0, i, 0)),
        pl.BlockSpec((_RB, H2), lambda i: (i, 0)),
        pl.BlockSpec((_RB, 1), lambda i: (i, 0)),
        pl.BlockSpec((H2,), lambda i: (0,)),
    ],
    out_specs=pl.BlockSpec((_RB, H2), lambda i: (i, 0)),
    out_shape=jax.ShapeDtypeStruct((N, H2), jnp.float32),
)


def kernel(x, edge_index, W1, b1, W2, b2):
    src = edge_index[0].astype(jnp.int32).reshape(NW, NCHUNK, CHUNK)
    dst = edge_index[1].astype(jnp.int32).reshape(NW, NCHUNK, CHUNK)

    W1p = jnp.pad(W1, ((0, 0), (0, H1P - H1)))
    b1p = jnp.pad(b1, (0, H1P - H1))
    W2p = jnp.pad(W2, ((0, H1P - H1), (0, 0)))

    zeros64 = jnp.zeros((NP, H1P), jnp.float32)
    zeros16 = jnp.zeros((NP, H2), jnp.float32)
    ones16t = jnp.ones((N, H2), jnp.float32)

    degp = _agg16(ones16t, src, dst, zeros16)
    hs1, dinv = _tc_a(x, W1p, degp)
    aggp1 = _agg64(hs1, src, dst, zeros64)
    hs2 = _tc_b(aggp1, hs1, dinv, W2p, b1p)
    aggp2 = _agg16(hs2, src, dst, zeros16)
    return _tc_c(aggp2, hs2, dinv, b2)


# speedup vs baseline: 39.1204x; 1.8599x over previous
import functools

import jax
import jax.numpy as jnp
from jax import lax
from jax.experimental import pallas as pl
from jax.experimental.pallas import tpu as pltpu
from jax.experimental.pallas import tpu_sc as plsc

N = 10000
E = 320000
D_IN = 128
H1 = 50
H1P = 64
H2 = 16

NC = 2
NS = 16
NW = NC * NS
CHUNK = 128
NCHUNK = 80
E_PAD = NW * NCHUNK * CHUNK
NP = 10240
RPW = NP // NS

_mesh = plsc.VectorSubcoreMesh(core_axis_name="c", subcore_axis_name="s")
_sc_params = pltpu.CompilerParams(use_tc_tiling_on_sc=False)


_DEG_K = 8


@functools.partial(
    pl.kernel,
    mesh=_mesh,
    out_type=jax.ShapeDtypeStruct((NC, NP, H2), jnp.float32),
    scratch_types=[
        pltpu.VMEM((NCHUNK, CHUNK), jnp.int32),
        pltpu.VMEM((CHUNK, H2), jnp.float32),
        pltpu.VMEM_SHARED((NP, H2), jnp.float32),
        pltpu.SemaphoreType.DMA,
    ],
    compiler_params=_sc_params,
)
def _deg_kernel(dst_hbm, degp_hbm, didx_v, ones_v, deg_sp, sem):
    c = lax.axis_index("c")
    s = lax.axis_index("s")
    wid = c * NS + s

    z16 = jnp.zeros((16,), jnp.float32)

    def zero_body(i, _):
        ones_v[i, :] = z16
        return ()

    lax.fori_loop(0, CHUNK, zero_body, ())
    for r in range(RPW // CHUNK):
        pltpu.sync_copy(ones_v, deg_sp.at[pl.ds(s * RPW + r * CHUNK, CHUNK)])

    o16 = jnp.ones((16,), jnp.float32)

    def ones_body(i, _):
        ones_v[i, :] = o16
        return ()

    lax.fori_loop(0, CHUNK, ones_body, ())
    pltpu.sync_copy(dst_hbm.at[wid], didx_v)
    plsc.subcore_barrier()

    def body(i, _):
        k0 = i * _DEG_K
        handles = [
            pltpu.async_copy(ones_v, deg_sp.at[didx_v.at[k0 + j]], sem, add=True)
            for j in range(_DEG_K)
        ]
        for h in handles:
            h.wait()
        return ()

    lax.fori_loop(0, NCHUNK // _DEG_K, body, ())

    plsc.subcore_barrier()
    pltpu.sync_copy(deg_sp.at[pl.ds(s * RPW, RPW)],
                    degp_hbm.at[c, pl.ds(s * RPW, RPW)])


def _make_agg_kernel(D):
    @functools.partial(
        pl.kernel,
        mesh=_mesh,
        out_type=jax.ShapeDtypeStruct((NC, NP, D), jnp.float32),
        scratch_types=[
            pltpu.VMEM((NCHUNK, CHUNK), jnp.int32),
            pltpu.VMEM((NCHUNK, CHUNK), jnp.int32),
            pltpu.VMEM((CHUNK, D), jnp.float32),
            pltpu.VMEM((CHUNK, D), jnp.float32),
            pltpu.VMEM_SHARED((NP, D), jnp.float32),
            pltpu.SemaphoreType.DMA,
            pltpu.SemaphoreType.DMA,
            pltpu.SemaphoreType.DMA,
            pltpu.SemaphoreType.DMA,
        ],
        compiler_params=_sc_params,
    )
    def agg(hs_hbm, src_hbm, dst_hbm, out_hbm,
            sidx_v, didx_v, rows_a, rows_b, agg_sp, ga, gb, sa, sb):
        c = lax.axis_index("c")
        s = lax.axis_index("s")
        wid = c * NS + s

        z16 = jnp.zeros((16,), jnp.float32)

        def zero_body(i, _):
            for j in range(D // 16):
                rows_a[i, pl.ds(j * 16, 16)] = z16
            return ()

        lax.fori_loop(0, CHUNK, zero_body, ())
        for r in range(RPW // CHUNK):
            pltpu.sync_copy(rows_a, agg_sp.at[pl.ds(s * RPW + r * CHUNK, CHUNK)])

        pltpu.sync_copy(src_hbm.at[wid], sidx_v)
        pltpu.sync_copy(dst_hbm.at[wid], didx_v)
        plsc.subcore_barrier()

        pltpu.async_copy(hs_hbm.at[sidx_v.at[0]], rows_a, ga)
        pltpu.async_copy(hs_hbm.at[sidx_v.at[1]], rows_b, gb)

        def body(i, _):
            k0 = 2 * i
            pltpu.make_async_copy(hs_hbm.at[sidx_v.at[k0]], rows_a, ga).wait()
            sc_a = pltpu.async_copy(rows_a, agg_sp.at[didx_v.at[k0]], sa,
                                    add=True)
            pltpu.make_async_copy(hs_hbm.at[sidx_v.at[k0 + 1]], rows_b,
                                  gb).wait()
            sc_b = pltpu.async_copy(rows_b, agg_sp.at[didx_v.at[k0 + 1]], sb,
                                    add=True)
            sc_a.wait()

            @pl.when(k0 + 2 < NCHUNK)
            def _():
                pltpu.async_copy(hs_hbm.at[sidx_v.at[k0 + 2]], rows_a, ga)

            sc_b.wait()

            @pl.when(k0 + 3 < NCHUNK)
            def _():
                pltpu.async_copy(hs_hbm.at[sidx_v.at[k0 + 3]], rows_b, gb)

            return ()

        lax.fori_loop(0, NCHUNK // 2, body, ())

        plsc.subcore_barrier()
        pltpu.sync_copy(agg_sp.at[pl.ds(s * RPW, RPW)],
                        out_hbm.at[c, pl.ds(s * RPW, RPW)])

    return agg


_agg64 = _make_agg_kernel(H1P)
_agg16 = _make_agg_kernel(H2)


_RB = 1000


def _tc_a_body(x_ref, w_ref, degp_ref, hs_ref, dinv_ref):
    deg = degp_ref[0, :, 0] + degp_ref[1, :, 0] + 1.0
    dinv = lax.rsqrt(deg)
    dinv_ref[...] = dinv[:, None]
    hs_ref[...] = jnp.dot(x_ref[...], w_ref[...],
                          preferred_element_type=jnp.float32) * dinv[:, None]


_tc_a = pl.pallas_call(
    _tc_a_body,
    grid=(N // _RB,),
    in_specs=[
        pl.BlockSpec((_RB, D_IN), lambda i: (i, 0)),
        pl.BlockSpec((D_IN, H1P), lambda i: (0, 0)),
        pl.BlockSpec((NC, _RB, H2), lambda i: (0, i, 0)),
    ],
    out_specs=[
        pl.BlockSpec((_RB, H1P), lambda i: (i, 0)),
        pl.BlockSpec((_RB, 1), lambda i: (i, 0)),
    ],
    out_shape=[
        jax.ShapeDtypeStruct((N, H1P), jnp.float32),
        jax.ShapeDtypeStruct((N, 1), jnp.float32),
    ],
)


def _tc_b_body(aggp_ref, hs1_ref, dinv_ref, w_ref, b_ref, hs2_ref):
    agg = aggp_ref[0] + aggp_ref[1] + hs1_ref[...]
    dinv = dinv_ref[...]
    h1 = jnp.maximum(agg * dinv + b_ref[...][None, :], 0.0)
    hs2_ref[...] = jnp.dot(h1, w_ref[...],
                           preferred_element_type=jnp.float32) * dinv


_tc_b = pl.pallas_call(
    _tc_b_body,
    grid=(N // _RB,),
    in_specs=[
        pl.BlockSpec((NC, _RB, H1P), lambda i: (0, i, 0)),
        pl.BlockSpec((_RB, H1P), lambda i: (i, 0)),
        pl.BlockSpec((_RB, 1), lambda i: (i, 0)),
        pl.BlockSpec((H1P, H2), lambda i: (0, 0)),
        pl.BlockSpec((H1P,), lambda i: (0,)),
    ],
    out_specs=pl.BlockSpec((_RB, H2), lambda i: (i, 0)),
    out_shape=jax.ShapeDtypeStruct((N, H2), jnp.float32),
)


def _tc_c_body(aggp_ref, hs2_ref, dinv_ref, b_ref, out_ref):
    agg = aggp_ref[0] + aggp_ref[1] + hs2_ref[...]
    h = agg * dinv_ref[...] + b_ref[...][None, :]
    m = jnp.max(h, axis=1, keepdims=True)
    lse = jnp.log(jnp.sum(jnp.exp(h - m), axis=1, keepdims=True))
    out_ref[...] = h - m - lse


_tc_c = pl.pallas_call(
    _tc_c_body,
    grid=(N // _RB,),
    in_specs=[
        pl.BlockSpec((NC, _RB, H2), lambda i: (0, i, 0)),
        pl.BlockSpec((_RB, H2), lambda i: (i, 0)),
        pl.BlockSpec((_RB, 1), lambda i: (i, 0)),
        pl.BlockSpec((H2,), lambda i: (0,)),
    ],
    out_specs=pl.BlockSpec((_RB, H2), lambda i: (i, 0)),
    out_shape=jax.ShapeDtypeStruct((N, H2), jnp.float32),
)


def kernel(x, edge_index, W1, b1, W2, b2):
    npad = E_PAD - E
    src_pad = jnp.arange(npad, dtype=jnp.int32) % N
    dst_pad = N + jnp.arange(npad, dtype=jnp.int32) % (NP - N)
    src = jnp.concatenate(
        [edge_index[0].astype(jnp.int32), src_pad]).reshape(NW, NCHUNK, CHUNK)
    dst = jnp.concatenate(
        [edge_index[1].astype(jnp.int32), dst_pad]).reshape(NW, NCHUNK, CHUNK)

    W1p = jnp.pad(W1, ((0, 0), (0, H1P - H1)))
    b1p = jnp.pad(b1, (0, H1P - H1))
    W2p = jnp.pad(W2, ((0, H1P - H1), (0, 0)))

    degp = _deg_kernel(dst)
    hs1, dinv = _tc_a(x, W1p, degp)
    aggp1 = _agg64(hs1, src, dst)
    hs2 = _tc_b(aggp1, hs1, dinv, W2p, b1p)
    aggp2 = _agg16(hs2, src, dst)
    return _tc_c(aggp2, hs2, dinv, b2)


# speedup vs baseline: 47.1431x; 1.2051x over previous
import functools

import jax
import jax.numpy as jnp
from jax import lax
from jax.experimental import pallas as pl
from jax.experimental.pallas import tpu as pltpu
from jax.experimental.pallas import tpu_sc as plsc

N = 10000
E = 320000
D_IN = 128
H1 = 50
H1P = 64
H2 = 16

NC = 2
NS = 16
NW = NC * NS
CHUNK = 128
NCHUNK = 80
E_PAD = NW * NCHUNK * CHUNK
NP = 10240
RPW = NP // NS

_mesh = plsc.VectorSubcoreMesh(core_axis_name="c", subcore_axis_name="s")
_sc_params = pltpu.CompilerParams(use_tc_tiling_on_sc=False)


_DEG_K = 8


@functools.partial(
    pl.kernel,
    mesh=_mesh,
    out_type=jax.ShapeDtypeStruct((NC, NP, H2), jnp.float32),
    scratch_types=[
        pltpu.VMEM((NCHUNK, CHUNK), jnp.int32),
        pltpu.VMEM((CHUNK, H2), jnp.float32),
        pltpu.VMEM_SHARED((NP, H2), jnp.float32),
        pltpu.SemaphoreType.DMA,
    ],
    compiler_params=_sc_params,
)
def _deg_kernel(dst_hbm, degp_hbm, didx_v, ones_v, deg_sp, sem):
    c = lax.axis_index("c")
    s = lax.axis_index("s")
    wid = c * NS + s

    z16 = jnp.zeros((16,), jnp.float32)

    def zero_body(i, _):
        ones_v[i, :] = z16
        return ()

    lax.fori_loop(0, CHUNK, zero_body, ())
    for r in range(RPW // CHUNK):
        pltpu.sync_copy(ones_v, deg_sp.at[pl.ds(s * RPW + r * CHUNK, CHUNK)])

    o16 = jnp.ones((16,), jnp.float32)

    def ones_body(i, _):
        ones_v[i, :] = o16
        return ()

    lax.fori_loop(0, CHUNK, ones_body, ())
    pltpu.sync_copy(dst_hbm.at[wid], didx_v)
    plsc.subcore_barrier()

    def body(i, _):
        k0 = i * _DEG_K
        handles = [
            pltpu.async_copy(ones_v, deg_sp.at[didx_v.at[k0 + j]], sem, add=True)
            for j in range(_DEG_K)
        ]
        for h in handles:
            h.wait()
        return ()

    lax.fori_loop(0, NCHUNK // _DEG_K, body, ())

    plsc.subcore_barrier()
    pltpu.sync_copy(deg_sp.at[pl.ds(s * RPW, RPW)],
                    degp_hbm.at[c, pl.ds(s * RPW, RPW)])


_NBUF = 4


def _make_agg_kernel(D, stage_hs):
    @functools.partial(
        pl.kernel,
        mesh=_mesh,
        out_type=jax.ShapeDtypeStruct((NC, NP, D), jnp.float32),
        scratch_types=[
            pltpu.VMEM((NCHUNK, CHUNK), jnp.int32),
            pltpu.VMEM((NCHUNK, CHUNK), jnp.int32),
            [pltpu.VMEM((CHUNK, D), jnp.float32)] * _NBUF,
            pltpu.VMEM_SHARED((NP, D), jnp.float32) if stage_hs else None,
            pltpu.VMEM_SHARED((NP, D), jnp.float32),
            [pltpu.SemaphoreType.DMA] * _NBUF,
            [pltpu.SemaphoreType.DMA] * _NBUF,
        ],
        compiler_params=_sc_params,
    )
    def agg(hs_hbm, src_hbm, dst_hbm, out_hbm,
            sidx_v, didx_v, rows, hs_sp_opt, agg_sp, gsem, ssem):
        c = lax.axis_index("c")
        s = lax.axis_index("s")
        wid = c * NS + s

        if stage_hs:
            pltpu.sync_copy(hs_hbm.at[pl.ds(s * RPW, RPW)],
                            hs_sp_opt.at[pl.ds(s * RPW, RPW)])
            hs_sp = hs_sp_opt
        else:
            hs_sp = hs_hbm

        z16 = jnp.zeros((16,), jnp.float32)

        def zero_body(i, _):
            for j in range(D // 16):
                rows[0][i, pl.ds(j * 16, 16)] = z16
            return ()

        lax.fori_loop(0, CHUNK, zero_body, ())
        for r in range(RPW // CHUNK):
            pltpu.sync_copy(rows[0], agg_sp.at[pl.ds(s * RPW + r * CHUNK, CHUNK)])

        pltpu.sync_copy(src_hbm.at[wid], sidx_v)
        pltpu.sync_copy(dst_hbm.at[wid], didx_v)
        plsc.subcore_barrier()

        for b in range(_NBUF):
            pltpu.async_copy(hs_sp.at[sidx_v.at[b]], rows[b], gsem[b])

        def body(i, _):
            k0 = i * _NBUF
            scs = []
            for b in range(_NBUF):
                pltpu.make_async_copy(hs_sp.at[sidx_v.at[k0 + b]], rows[b],
                                      gsem[b]).wait()
                scs.append(pltpu.async_copy(rows[b],
                                            agg_sp.at[didx_v.at[k0 + b]],
                                            ssem[b], add=True))
            for b in range(_NBUF):
                scs[b].wait()
                kn = k0 + _NBUF + b

                @pl.when(kn < NCHUNK)
                def _(b=b, kn=kn):
                    pltpu.async_copy(hs_sp.at[sidx_v.at[kn]], rows[b], gsem[b])

            return ()

        lax.fori_loop(0, NCHUNK // _NBUF, body, ())

        plsc.subcore_barrier()
        pltpu.sync_copy(agg_sp.at[pl.ds(s * RPW, RPW)],
                        out_hbm.at[c, pl.ds(s * RPW, RPW)])

    return agg


_agg64 = _make_agg_kernel(H1P, stage_hs=False)
_agg16 = _make_agg_kernel(H2, stage_hs=True)


_RB = 1000


def _tc_a_body(x_ref, w_ref, degp_ref, hs_ref, dinv_ref):
    deg = degp_ref[0, :, 0] + degp_ref[1, :, 0] + 1.0
    dinv = lax.rsqrt(deg)
    dinv_ref[...] = dinv[:, None]
    hs_ref[...] = jnp.dot(x_ref[...], w_ref[...],
                          preferred_element_type=jnp.float32) * dinv[:, None]


_tc_a = pl.pallas_call(
    _tc_a_body,
    grid=(N // _RB,),
    in_specs=[
        pl.BlockSpec((_RB, D_IN), lambda i: (i, 0)),
        pl.BlockSpec((D_IN, H1P), lambda i: (0, 0)),
        pl.BlockSpec((NC, _RB, H2), lambda i: (0, i, 0)),
    ],
    out_specs=[
        pl.BlockSpec((_RB, H1P), lambda i: (i, 0)),
        pl.BlockSpec((_RB, 1), lambda i: (i, 0)),
    ],
    out_shape=[
        jax.ShapeDtypeStruct((NP, H1P), jnp.float32),
        jax.ShapeDtypeStruct((N, 1), jnp.float32),
    ],
)


def _tc_b_body(aggp_ref, hs1_ref, dinv_ref, w_ref, b_ref, hs2_ref):
    agg = aggp_ref[0] + aggp_ref[1] + hs1_ref[...]
    dinv = dinv_ref[...]
    h1 = jnp.maximum(agg * dinv + b_ref[...][None, :], 0.0)
    hs2_ref[...] = jnp.dot(h1, w_ref[...],
                           preferred_element_type=jnp.float32) * dinv


_tc_b = pl.pallas_call(
    _tc_b_body,
    grid=(N // _RB,),
    in_specs=[
        pl.BlockSpec((NC, _RB, H1P), lambda i: (0, i, 0)),
        pl.BlockSpec((_RB, H1P), lambda i: (i, 0)),
        pl.BlockSpec((_RB, 1), lambda i: (i, 0)),
        pl.BlockSpec((H1P, H2), lambda i: (0, 0)),
        pl.BlockSpec((H1P,), lambda i: (0,)),
    ],
    out_specs=pl.BlockSpec((_RB, H2), lambda i: (i, 0)),
    out_shape=jax.ShapeDtypeStruct((NP, H2), jnp.float32),
)


def _tc_c_body(aggp_ref, hs2_ref, dinv_ref, b_ref, out_ref):
    agg = aggp_ref[0] + aggp_ref[1] + hs2_ref[...]
    h = agg * dinv_ref[...] + b_ref[...][None, :]
    m = jnp.max(h, axis=1, keepdims=True)
    lse = jnp.log(jnp.sum(jnp.exp(h - m), axis=1, keepdims=True))
    out_ref[...] = h - m - lse


_tc_c = pl.pallas_call(
    _tc_c_body,
    grid=(N // _RB,),
    in_specs=[
        pl.BlockSpec((NC, _RB, H2), lambda i: (0, i, 0)),
        pl.BlockSpec((_RB, H2), lambda i: (i, 0)),
        pl.BlockSpec((_RB, 1), lambda i: (i, 0)),
        pl.BlockSpec((H2,), lambda i: (0,)),
    ],
    out_specs=pl.BlockSpec((_RB, H2), lambda i: (i, 0)),
    out_shape=jax.ShapeDtypeStruct((N, H2), jnp.float32),
)


def kernel(x, edge_index, W1, b1, W2, b2):
    npad = E_PAD - E
    src_pad = jnp.arange(npad, dtype=jnp.int32) % N
    dst_pad = N + jnp.arange(npad, dtype=jnp.int32) % (NP - N)
    src = jnp.concatenate(
        [edge_index[0].astype(jnp.int32), src_pad]).reshape(NW, NCHUNK, CHUNK)
    dst = jnp.concatenate(
        [edge_index[1].astype(jnp.int32), dst_pad]).reshape(NW, NCHUNK, CHUNK)

    W1p = jnp.pad(W1, ((0, 0), (0, H1P - H1)))
    b1p = jnp.pad(b1, (0, H1P - H1))
    W2p = jnp.pad(W2, ((0, H1P - H1), (0, 0)))

    degp = _deg_kernel(dst)
    hs1, dinv = _tc_a(x, W1p, degp)
    aggp1 = _agg64(hs1, src, dst)
    hs2 = _tc_b(aggp1, hs1, dinv, W2p, b1p)
    aggp2 = _agg16(hs2, src, dst)
    return _tc_c(aggp2, hs2, dinv, b2)


# speedup vs baseline: 48.6732x; 1.0325x over previous
import functools

import jax
import jax.numpy as jnp
from jax import lax
from jax.experimental import pallas as pl
from jax.experimental.pallas import tpu as pltpu
from jax.experimental.pallas import tpu_sc as plsc

N = 10000
E = 320000
D_IN = 128
H1 = 50
H1P = 64
H2 = 16

NC = 2
NS = 16
NW = NC * NS
CHUNK = 128
NCHUNK = 80
E_PAD = NW * NCHUNK * CHUNK
NP = 10240
RPW = NP // NS

_mesh = plsc.VectorSubcoreMesh(core_axis_name="c", subcore_axis_name="s")
_sc_params = pltpu.CompilerParams(use_tc_tiling_on_sc=False)


_DEG_K = 8


@functools.partial(
    pl.kernel,
    mesh=_mesh,
    out_type=jax.ShapeDtypeStruct((NC, NP, H2), jnp.float32),
    scratch_types=[
        pltpu.VMEM((NCHUNK, CHUNK), jnp.int32),
        pltpu.VMEM((CHUNK, H2), jnp.float32),
        pltpu.VMEM_SHARED((NP, H2), jnp.float32),
        pltpu.SemaphoreType.DMA,
    ],
    compiler_params=_sc_params,
)
def _deg_kernel(dst_hbm, degp_hbm, didx_v, ones_v, deg_sp, sem):
    c = lax.axis_index("c")
    s = lax.axis_index("s")
    wid = c * NS + s

    z16 = jnp.zeros((16,), jnp.float32)

    def zero_body(i, _):
        ones_v[i, :] = z16
        return ()

    lax.fori_loop(0, CHUNK, zero_body, ())
    for r in range(RPW // CHUNK):
        pltpu.sync_copy(ones_v, deg_sp.at[pl.ds(s * RPW + r * CHUNK, CHUNK)])

    o16 = jnp.ones((16,), jnp.float32)

    def ones_body(i, _):
        ones_v[i, :] = o16
        return ()

    lax.fori_loop(0, CHUNK, ones_body, ())
    pltpu.sync_copy(dst_hbm.at[wid], didx_v)
    plsc.subcore_barrier()

    def body(i, _):
        k0 = i * _DEG_K
        handles = [
            pltpu.async_copy(ones_v, deg_sp.at[didx_v.at[k0 + j]], sem, add=True)
            for j in range(_DEG_K)
        ]
        for h in handles:
            h.wait()
        return ()

    lax.fori_loop(0, NCHUNK // _DEG_K, body, ())

    plsc.subcore_barrier()
    pltpu.sync_copy(deg_sp.at[pl.ds(s * RPW, RPW)],
                    degp_hbm.at[c, pl.ds(s * RPW, RPW)])


_NBUF = 4


def _make_agg_kernel(D, stage_hs):
    @functools.partial(
        pl.kernel,
        mesh=_mesh,
        out_type=jax.ShapeDtypeStruct((NC, NP, D), jnp.float32),
        scratch_types=[
            pltpu.VMEM((NCHUNK, CHUNK), jnp.int32),
            pltpu.VMEM((NCHUNK, CHUNK), jnp.int32),
            [pltpu.VMEM((CHUNK, D), jnp.float32)] * _NBUF,
            pltpu.VMEM_SHARED((NP, D), jnp.float32) if stage_hs else None,
            pltpu.VMEM_SHARED((NP, D), jnp.float32),
            [pltpu.SemaphoreType.DMA] * _NBUF,
            [pltpu.SemaphoreType.DMA] * _NBUF,
        ],
        compiler_params=_sc_params,
    )
    def agg(hs_hbm, src_hbm, dst_hbm, out_hbm,
            sidx_v, didx_v, rows, hs_sp_opt, agg_sp, gsem, ssem):
        c = lax.axis_index("c")
        s = lax.axis_index("s")
        wid = c * NS + s

        if stage_hs:
            pltpu.sync_copy(hs_hbm.at[pl.ds(s * RPW, RPW)],
                            hs_sp_opt.at[pl.ds(s * RPW, RPW)])
            hs_sp = hs_sp_opt
        else:
            hs_sp = hs_hbm

        z16 = jnp.zeros((16,), jnp.float32)

        def zero_body(i, _):
            for j in range(D // 16):
                rows[0][i, pl.ds(j * 16, 16)] = z16
            return ()

        lax.fori_loop(0, CHUNK, zero_body, ())
        for r in range(RPW // CHUNK):
            pltpu.sync_copy(rows[0], agg_sp.at[pl.ds(s * RPW + r * CHUNK, CHUNK)])

        pltpu.sync_copy(src_hbm.at[wid], sidx_v)
        pltpu.sync_copy(dst_hbm.at[wid], didx_v)
        plsc.subcore_barrier()

        for b in range(_NBUF):
            pltpu.async_copy(hs_sp.at[sidx_v.at[b]], rows[b], gsem[b])

        def body(i, _):
            k0 = i * _NBUF
            scs = []
            for b in range(_NBUF):
                pltpu.make_async_copy(hs_sp.at[sidx_v.at[k0 + b]], rows[b],
                                      gsem[b]).wait()
                scs.append(pltpu.async_copy(rows[b],
                                            agg_sp.at[didx_v.at[k0 + b]],
                                            ssem[b], add=True))
            for b in range(_NBUF):
                scs[b].wait()
                kn = k0 + _NBUF + b

                @pl.when(kn < NCHUNK)
                def _(b=b, kn=kn):
                    pltpu.async_copy(hs_sp.at[sidx_v.at[kn]], rows[b], gsem[b])

            return ()

        lax.fori_loop(0, NCHUNK // _NBUF, body, ())

        plsc.subcore_barrier()
        pltpu.sync_copy(agg_sp.at[pl.ds(s * RPW, RPW)],
                        out_hbm.at[c, pl.ds(s * RPW, RPW)])

    return agg


_agg64 = _make_agg_kernel(H1P, stage_hs=False)
_agg16 = _make_agg_kernel(H2, stage_hs=True)


_RB = 2000


def _tc_a_body(x_ref, w_ref, degp_ref, hs_ref, dinv_ref):
    deg = degp_ref[0, :, 0] + degp_ref[1, :, 0] + 1.0
    dinv = lax.rsqrt(deg)
    dinv_ref[...] = dinv[:, None]
    hs_ref[...] = jnp.dot(x_ref[...], w_ref[...],
                          preferred_element_type=jnp.float32) * dinv[:, None]


_tc_a = pl.pallas_call(
    _tc_a_body,
    grid=(N // _RB,),
    in_specs=[
        pl.BlockSpec((_RB, D_IN), lambda i: (i, 0)),
        pl.BlockSpec((D_IN, H1P), lambda i: (0, 0)),
        pl.BlockSpec((NC, _RB, H2), lambda i: (0, i, 0)),
    ],
    out_specs=[
        pl.BlockSpec((_RB, H1P), lambda i: (i, 0)),
        pl.BlockSpec((_RB, 1), lambda i: (i, 0)),
    ],
    out_shape=[
        jax.ShapeDtypeStruct((NP, H1P), jnp.float32),
        jax.ShapeDtypeStruct((N, 1), jnp.float32),
    ],
)


def _tc_b_body(aggp_ref, hs1_ref, dinv_ref, w_ref, b_ref, hs2_ref):
    agg = aggp_ref[0] + aggp_ref[1] + hs1_ref[...]
    dinv = dinv_ref[...]
    h1 = jnp.maximum(agg * dinv + b_ref[...][None, :], 0.0)
    hs2_ref[...] = jnp.dot(h1, w_ref[...],
                           preferred_element_type=jnp.float32) * dinv


_tc_b = pl.pallas_call(
    _tc_b_body,
    grid=(N // _RB,),
    in_specs=[
        pl.BlockSpec((NC, _RB, H1P), lambda i: (0, i, 0)),
        pl.BlockSpec((_RB, H1P), lambda i: (i, 0)),
        pl.BlockSpec((_RB, 1), lambda i: (i, 0)),
        pl.BlockSpec((H1P, H2), lambda i: (0, 0)),
        pl.BlockSpec((H1P,), lambda i: (0,)),
    ],
    out_specs=pl.BlockSpec((_RB, H2), lambda i: (i, 0)),
    out_shape=jax.ShapeDtypeStruct((NP, H2), jnp.float32),
)


def _tc_c_body(aggp_ref, hs2_ref, dinv_ref, b_ref, out_ref):
    agg = aggp_ref[0] + aggp_ref[1] + hs2_ref[...]
    h = agg * dinv_ref[...] + b_ref[...][None, :]
    m = jnp.max(h, axis=1, keepdims=True)
    lse = jnp.log(jnp.sum(jnp.exp(h - m), axis=1, keepdims=True))
    out_ref[...] = h - m - lse


_tc_c = pl.pallas_call(
    _tc_c_body,
    grid=(N // _RB,),
    in_specs=[
        pl.BlockSpec((NC, _RB, H2), lambda i: (0, i, 0)),
        pl.BlockSpec((_RB, H2), lambda i: (i, 0)),
        pl.BlockSpec((_RB, 1), lambda i: (i, 0)),
        pl.BlockSpec((H2,), lambda i: (0,)),
    ],
    out_specs=pl.BlockSpec((_RB, H2), lambda i: (i, 0)),
    out_shape=jax.ShapeDtypeStruct((N, H2), jnp.float32),
)


def kernel(x, edge_index, W1, b1, W2, b2):
    npad = E_PAD - E
    src_pad = jnp.arange(npad, dtype=jnp.int32) % N
    dst_pad = N + jnp.arange(npad, dtype=jnp.int32) % (NP - N)
    src = jnp.concatenate(
        [edge_index[0].astype(jnp.int32), src_pad]).reshape(NW, NCHUNK, CHUNK)
    dst = jnp.concatenate(
        [edge_index[1].astype(jnp.int32), dst_pad]).reshape(NW, NCHUNK, CHUNK)

    W1p = jnp.pad(W1, ((0, 0), (0, H1P - H1)))
    b1p = jnp.pad(b1, (0, H1P - H1))
    W2p = jnp.pad(W2, ((0, H1P - H1), (0, 0)))

    degp = _deg_kernel(dst)
    hs1, dinv = _tc_a(x, W1p, degp)
    aggp1 = _agg64(hs1, src, dst)
    hs2 = _tc_b(aggp1, hs1, dinv, W2p, b1p)
    aggp2 = _agg16(hs2, src, dst)
    return _tc_c(aggp2, hs2, dinv, b2)


# speedup vs baseline: 49.8689x; 1.0246x over previous
import functools

import jax
import jax.numpy as jnp
from jax import lax
from jax.experimental import pallas as pl
from jax.experimental.pallas import tpu as pltpu
from jax.experimental.pallas import tpu_sc as plsc

N = 10000
E = 320000
D_IN = 128
H1 = 50
H1P = 64
H2 = 16

NC = 2
NS = 16
NW = NC * NS
CHUNK = 128
NCHUNK = 80
E_PAD = NW * NCHUNK * CHUNK
NP = 10240
RPW = NP // NS

_mesh = plsc.VectorSubcoreMesh(core_axis_name="c", subcore_axis_name="s")
_sc_params = pltpu.CompilerParams(use_tc_tiling_on_sc=False)


_DEG_K = 8


@functools.partial(
    pl.kernel,
    mesh=_mesh,
    out_type=jax.ShapeDtypeStruct((NC, NP, H2), jnp.float32),
    scratch_types=[
        pltpu.VMEM((NCHUNK, CHUNK), jnp.int32),
        pltpu.VMEM((CHUNK, H2), jnp.float32),
        pltpu.VMEM_SHARED((NP, H2), jnp.float32),
        pltpu.SemaphoreType.DMA,
    ],
    compiler_params=_sc_params,
)
def _deg_kernel(dst_hbm, degp_hbm, didx_v, ones_v, deg_sp, sem):
    c = lax.axis_index("c")
    s = lax.axis_index("s")
    wid = c * NS + s

    z16 = jnp.zeros((16,), jnp.float32)

    def zero_body(i, _):
        ones_v[i, :] = z16
        return ()

    lax.fori_loop(0, CHUNK, zero_body, ())
    for r in range(RPW // CHUNK):
        pltpu.sync_copy(ones_v, deg_sp.at[pl.ds(s * RPW + r * CHUNK, CHUNK)])

    o16 = jnp.ones((16,), jnp.float32)

    def ones_body(i, _):
        ones_v[i, :] = o16
        return ()

    lax.fori_loop(0, CHUNK, ones_body, ())
    pltpu.sync_copy(dst_hbm.at[wid], didx_v)
    plsc.subcore_barrier()

    def body(i, _):
        k0 = i * _DEG_K
        handles = [
            pltpu.async_copy(ones_v, deg_sp.at[didx_v.at[k0 + j]], sem, add=True)
            for j in range(_DEG_K)
        ]
        for h in handles:
            h.wait()
        return ()

    lax.fori_loop(0, NCHUNK // _DEG_K, body, ())

    plsc.subcore_barrier()
    pltpu.sync_copy(deg_sp.at[pl.ds(s * RPW, RPW)],
                    degp_hbm.at[c, pl.ds(s * RPW, RPW)])


_NBUF = 8


def _make_agg_kernel(D, stage_hs):
    @functools.partial(
        pl.kernel,
        mesh=_mesh,
        out_type=jax.ShapeDtypeStruct((NC, NP, D), jnp.float32),
        scratch_types=[
            pltpu.VMEM((NCHUNK, CHUNK), jnp.int32),
            pltpu.VMEM((NCHUNK, CHUNK), jnp.int32),
            [pltpu.VMEM((CHUNK, D), jnp.float32)] * _NBUF,
            pltpu.VMEM_SHARED((NP, D), jnp.float32) if stage_hs else None,
            pltpu.VMEM_SHARED((NP, D), jnp.float32),
            [pltpu.SemaphoreType.DMA] * _NBUF,
            [pltpu.SemaphoreType.DMA] * _NBUF,
        ],
        compiler_params=_sc_params,
    )
    def agg(hs_hbm, src_hbm, dst_hbm, out_hbm,
            sidx_v, didx_v, rows, hs_sp_opt, agg_sp, gsem, ssem):
        c = lax.axis_index("c")
        s = lax.axis_index("s")
        wid = c * NS + s

        if stage_hs:
            pltpu.sync_copy(hs_hbm.at[pl.ds(s * RPW, RPW)],
                            hs_sp_opt.at[pl.ds(s * RPW, RPW)])
            hs_sp = hs_sp_opt
        else:
            hs_sp = hs_hbm

        z16 = jnp.zeros((16,), jnp.float32)

        def zero_body(i, _):
            for j in range(D // 16):
                rows[0][i, pl.ds(j * 16, 16)] = z16
            return ()

        lax.fori_loop(0, CHUNK, zero_body, ())
        for r in range(RPW // CHUNK):
            pltpu.sync_copy(rows[0], agg_sp.at[pl.ds(s * RPW + r * CHUNK, CHUNK)])

        pltpu.sync_copy(src_hbm.at[wid], sidx_v)
        pltpu.sync_copy(dst_hbm.at[wid], didx_v)
        plsc.subcore_barrier()

        for b in range(_NBUF):
            pltpu.async_copy(hs_sp.at[sidx_v.at[b]], rows[b], gsem[b])

        def body(i, _):
            k0 = i * _NBUF
            scs = []
            for b in range(_NBUF):
                pltpu.make_async_copy(hs_sp.at[sidx_v.at[k0 + b]], rows[b],
                                      gsem[b]).wait()
                scs.append(pltpu.async_copy(rows[b],
                                            agg_sp.at[didx_v.at[k0 + b]],
                                            ssem[b], add=True))
            for b in range(_NBUF):
                scs[b].wait()
                kn = k0 + _NBUF + b

                @pl.when(kn < NCHUNK)
                def _(b=b, kn=kn):
                    pltpu.async_copy(hs_sp.at[sidx_v.at[kn]], rows[b], gsem[b])

            return ()

        lax.fori_loop(0, NCHUNK // _NBUF, body, ())

        plsc.subcore_barrier()
        pltpu.sync_copy(agg_sp.at[pl.ds(s * RPW, RPW)],
                        out_hbm.at[c, pl.ds(s * RPW, RPW)])

    return agg


_agg64 = _make_agg_kernel(H1P, stage_hs=False)
_agg16 = _make_agg_kernel(H2, stage_hs=True)


_RB = 2000


def _tc_a_body(x_ref, w_ref, degp_ref, hs_ref, dinv_ref):
    deg = degp_ref[0, :, 0] + degp_ref[1, :, 0] + 1.0
    dinv = lax.rsqrt(deg)
    dinv_ref[...] = dinv[:, None]
    hs_ref[...] = jnp.dot(x_ref[...], w_ref[...],
                          preferred_element_type=jnp.float32) * dinv[:, None]


_tc_a = pl.pallas_call(
    _tc_a_body,
    grid=(N // _RB,),
    in_specs=[
        pl.BlockSpec((_RB, D_IN), lambda i: (i, 0)),
        pl.BlockSpec((D_IN, H1P), lambda i: (0, 0)),
        pl.BlockSpec((NC, _RB, H2), lambda i: (0, i, 0)),
    ],
    out_specs=[
        pl.BlockSpec((_RB, H1P), lambda i: (i, 0)),
        pl.BlockSpec((_RB, 1), lambda i: (i, 0)),
    ],
    out_shape=[
        jax.ShapeDtypeStruct((NP, H1P), jnp.float32),
        jax.ShapeDtypeStruct((N, 1), jnp.float32),
    ],
)


def _tc_b_body(aggp_ref, hs1_ref, dinv_ref, w_ref, b_ref, hs2_ref):
    agg = aggp_ref[0] + aggp_ref[1] + hs1_ref[...]
    dinv = dinv_ref[...]
    h1 = jnp.maximum(agg * dinv + b_ref[...][None, :], 0.0)
    hs2_ref[...] = jnp.dot(h1, w_ref[...],
                           preferred_element_type=jnp.float32) * dinv


_tc_b = pl.pallas_call(
    _tc_b_body,
    grid=(N // _RB,),
    in_specs=[
        pl.BlockSpec((NC, _RB, H1P), lambda i: (0, i, 0)),
        pl.BlockSpec((_RB, H1P), lambda i: (i, 0)),
        pl.BlockSpec((_RB, 1), lambda i: (i, 0)),
        pl.BlockSpec((H1P, H2), lambda i: (0, 0)),
        pl.BlockSpec((H1P,), lambda i: (0,)),
    ],
    out_specs=pl.BlockSpec((_RB, H2), lambda i: (i, 0)),
    out_shape=jax.ShapeDtypeStruct((NP, H2), jnp.float32),
)


def _tc_c_body(aggp_ref, hs2_ref, dinv_ref, b_ref, out_ref):
    agg = aggp_ref[0] + aggp_ref[1] + hs2_ref[...]
    h = agg * dinv_ref[...] + b_ref[...][None, :]
    m = jnp.max(h, axis=1, keepdims=True)
    lse = jnp.log(jnp.sum(jnp.exp(h - m), axis=1, keepdims=True))
    out_ref[...] = h - m - lse


_tc_c = pl.pallas_call(
    _tc_c_body,
    grid=(N // _RB,),
    in_specs=[
        pl.BlockSpec((NC, _RB, H2), lambda i: (0, i, 0)),
        pl.BlockSpec((_RB, H2), lambda i: (i, 0)),
        pl.BlockSpec((_RB, 1), lambda i: (i, 0)),
        pl.BlockSpec((H2,), lambda i: (0,)),
    ],
    out_specs=pl.BlockSpec((_RB, H2), lambda i: (i, 0)),
    out_shape=jax.ShapeDtypeStruct((N, H2), jnp.float32),
)


def kernel(x, edge_index, W1, b1, W2, b2):
    npad = E_PAD - E
    src_pad = jnp.arange(npad, dtype=jnp.int32) % N
    dst_pad = N + jnp.arange(npad, dtype=jnp.int32) % (NP - N)
    src = jnp.concatenate(
        [edge_index[0].astype(jnp.int32), src_pad]).reshape(NW, NCHUNK, CHUNK)
    dst = jnp.concatenate(
        [edge_index[1].astype(jnp.int32), dst_pad]).reshape(NW, NCHUNK, CHUNK)

    W1p = jnp.pad(W1, ((0, 0), (0, H1P - H1)))
    b1p = jnp.pad(b1, (0, H1P - H1))
    W2p = jnp.pad(W2, ((0, H1P - H1), (0, 0)))

    degp = _deg_kernel(dst)
    hs1, dinv = _tc_a(x, W1p, degp)
    aggp1 = _agg64(hs1, src, dst)
    hs2 = _tc_b(aggp1, hs1, dinv, W2p, b1p)
    aggp2 = _agg16(hs2, src, dst)
    return _tc_c(aggp2, hs2, dinv, b2)


# speedup vs baseline: 52.5157x; 1.0531x over previous
import functools

import jax
import jax.numpy as jnp
from jax import lax
from jax.experimental import pallas as pl
from jax.experimental.pallas import tpu as pltpu
from jax.experimental.pallas import tpu_sc as plsc

N = 10000
E = 320000
D_IN = 128
H1 = 50
H1P = 64
H2 = 16

NC = 2
NS = 16
NW = NC * NS
CHUNK = 128
NCHUNK = 80
E_PAD = NW * NCHUNK * CHUNK
NP = 10240
RPW = NP // NS

_mesh = plsc.VectorSubcoreMesh(core_axis_name="c", subcore_axis_name="s")
_sc_params = pltpu.CompilerParams(use_tc_tiling_on_sc=False)


_DEG_K = 8


@functools.partial(
    pl.kernel,
    mesh=_mesh,
    out_type=jax.ShapeDtypeStruct((NC, NP, H2), jnp.float32),
    scratch_types=[
        pltpu.VMEM((NCHUNK, CHUNK), jnp.int32),
        pltpu.VMEM((CHUNK, H2), jnp.float32),
        pltpu.VMEM_SHARED((NP, H2), jnp.float32),
        pltpu.SemaphoreType.DMA,
    ],
    compiler_params=_sc_params,
)
def _deg_kernel(dst_hbm, degp_hbm, didx_v, ones_v, deg_sp, sem):
    c = lax.axis_index("c")
    s = lax.axis_index("s")
    wid = c * NS + s

    z16 = jnp.zeros((16,), jnp.float32)

    def zero_body(i, _):
        ones_v[i, :] = z16
        return ()

    lax.fori_loop(0, CHUNK, zero_body, ())
    for r in range(RPW // CHUNK):
        pltpu.sync_copy(ones_v, deg_sp.at[pl.ds(s * RPW + r * CHUNK, CHUNK)])

    o16 = jnp.ones((16,), jnp.float32)

    def ones_body(i, _):
        ones_v[i, :] = o16
        return ()

    lax.fori_loop(0, CHUNK, ones_body, ())
    pltpu.sync_copy(dst_hbm.at[wid], didx_v)
    plsc.subcore_barrier()

    def body(i, _):
        k0 = i * _DEG_K
        handles = [
            pltpu.async_copy(ones_v, deg_sp.at[didx_v.at[k0 + j]], sem, add=True)
            for j in range(_DEG_K)
        ]
        for h in handles:
            h.wait()
        return ()

    lax.fori_loop(0, NCHUNK // _DEG_K, body, ())

    plsc.subcore_barrier()
    pltpu.sync_copy(deg_sp.at[pl.ds(s * RPW, RPW)],
                    degp_hbm.at[c, pl.ds(s * RPW, RPW)])


_NBUF = 8


def _make_agg_kernel(D, stage_hs):
    @functools.partial(
        pl.kernel,
        mesh=_mesh,
        out_type=jax.ShapeDtypeStruct((NC, NP, D), jnp.float32),
        scratch_types=[
            pltpu.VMEM((NCHUNK, CHUNK), jnp.int32),
            pltpu.VMEM((NCHUNK, CHUNK), jnp.int32),
            [pltpu.VMEM((CHUNK, D), jnp.float32)] * _NBUF,
            pltpu.VMEM_SHARED((NP, D), jnp.float32) if stage_hs else None,
            pltpu.VMEM_SHARED((NP, D), jnp.float32),
            [pltpu.SemaphoreType.DMA] * _NBUF,
            [pltpu.SemaphoreType.DMA] * _NBUF,
        ],
        compiler_params=_sc_params,
    )
    def agg(hs_hbm, src_hbm, dst_hbm, out_hbm,
            sidx_v, didx_v, rows, hs_sp_opt, agg_sp, gsem, ssem):
        c = lax.axis_index("c")
        s = lax.axis_index("s")
        wid = c * NS + s

        if stage_hs:
            pltpu.sync_copy(hs_hbm.at[pl.ds(s * RPW, RPW)],
                            hs_sp_opt.at[pl.ds(s * RPW, RPW)])
            hs_sp = hs_sp_opt
        else:
            hs_sp = hs_hbm

        @pl.when(c == 0)
        def _():
            pltpu.sync_copy(hs_hbm.at[pl.ds(s * RPW, RPW)],
                            agg_sp.at[pl.ds(s * RPW, RPW)])

        @pl.when(c == 1)
        def _():
            z16 = jnp.zeros((16,), jnp.float32)

            def zero_body(i, _):
                for j in range(D // 16):
                    rows[0][i, pl.ds(j * 16, 16)] = z16
                return ()

            lax.fori_loop(0, CHUNK, zero_body, ())
            for r in range(RPW // CHUNK):
                pltpu.sync_copy(rows[0],
                                agg_sp.at[pl.ds(s * RPW + r * CHUNK, CHUNK)])

        pltpu.sync_copy(src_hbm.at[wid], sidx_v)
        pltpu.sync_copy(dst_hbm.at[wid], didx_v)
        plsc.subcore_barrier()

        for b in range(_NBUF):
            pltpu.async_copy(hs_sp.at[sidx_v.at[b]], rows[b], gsem[b])

        def body(i, _):
            k0 = i * _NBUF
            scs = []
            for b in range(_NBUF):
                pltpu.make_async_copy(hs_sp.at[sidx_v.at[k0 + b]], rows[b],
                                      gsem[b]).wait()
                scs.append(pltpu.async_copy(rows[b],
                                            agg_sp.at[didx_v.at[k0 + b]],
                                            ssem[b], add=True))
            for b in range(_NBUF):
                scs[b].wait()
                kn = k0 + _NBUF + b

                @pl.when(kn < NCHUNK)
                def _(b=b, kn=kn):
                    pltpu.async_copy(hs_sp.at[sidx_v.at[kn]], rows[b], gsem[b])

            return ()

        lax.fori_loop(0, NCHUNK // _NBUF, body, ())

        plsc.subcore_barrier()
        pltpu.sync_copy(agg_sp.at[pl.ds(s * RPW, RPW)],
                        out_hbm.at[c, pl.ds(s * RPW, RPW)])

    return agg


_agg64 = _make_agg_kernel(H1P, stage_hs=False)
_agg16 = _make_agg_kernel(H2, stage_hs=True)


_RB = 2000


def _tc_a_body(x_ref, w_ref, degp_ref, hs_ref, dinv_ref):
    deg = degp_ref[0, :, 0] + degp_ref[1, :, 0] + 1.0
    dinv = lax.rsqrt(deg)
    dinv_ref[...] = dinv[:, None]
    hs_ref[...] = jnp.dot(x_ref[...], w_ref[...],
                          preferred_element_type=jnp.float32) * dinv[:, None]


_tc_a = pl.pallas_call(
    _tc_a_body,
    grid=(N // _RB,),
    in_specs=[
        pl.BlockSpec((_RB, D_IN), lambda i: (i, 0)),
        pl.BlockSpec((D_IN, H1P), lambda i: (0, 0)),
        pl.BlockSpec((NC, _RB, H2), lambda i: (0, i, 0)),
    ],
    out_specs=[
        pl.BlockSpec((_RB, H1P), lambda i: (i, 0)),
        pl.BlockSpec((_RB, 1), lambda i: (i, 0)),
    ],
    out_shape=[
        jax.ShapeDtypeStruct((NP, H1P), jnp.float32),
        jax.ShapeDtypeStruct((N, 1), jnp.float32),
    ],
)


def _tc_b_body(aggp_ref, dinv_ref, w_ref, b_ref, hs2_ref):
    agg = aggp_ref[0] + aggp_ref[1]
    dinv = dinv_ref[...]
    h1 = jnp.maximum(agg * dinv + b_ref[...][None, :], 0.0)
    hs2_ref[...] = jnp.dot(h1, w_ref[...],
                           preferred_element_type=jnp.float32) * dinv


_tc_b = pl.pallas_call(
    _tc_b_body,
    grid=(N // _RB,),
    in_specs=[
        pl.BlockSpec((NC, _RB, H1P), lambda i: (0, i, 0)),
        pl.BlockSpec((_RB, 1), lambda i: (i, 0)),
        pl.BlockSpec((H1P, H2), lambda i: (0, 0)),
        pl.BlockSpec((H1P,), lambda i: (0,)),
    ],
    out_specs=pl.BlockSpec((_RB, H2), lambda i: (i, 0)),
    out_shape=jax.ShapeDtypeStruct((NP, H2), jnp.float32),
)


_RI = 256


def _tc_c_body(aggp_ref, degp_ref, b_ref, out_ref):
    agg = aggp_ref[0] + aggp_ref[1]
    deg = degp_ref[0] + degp_ref[1] + 1.0
    h = agg * lax.rsqrt(deg) + b_ref[...][None, :]
    m = jnp.max(h, axis=1, keepdims=True)
    ex = jnp.exp(h - m)
    seg_i = lax.broadcasted_iota(jnp.int32, (128, 8), 0) // 16
    seg_j = lax.broadcasted_iota(jnp.int32, (128, 8), 1)
    e8 = (seg_i == seg_j).astype(jnp.float32)
    seg_i2 = lax.broadcasted_iota(jnp.int32, (8, 128), 0)
    seg_j2 = lax.broadcasted_iota(jnp.int32, (8, 128), 1) // 16
    e8t = (seg_i2 == seg_j2).astype(jnp.float32)
    segsum = jnp.dot(ex, e8, preferred_element_type=jnp.float32,
                     precision=lax.Precision.HIGHEST)
    lse = jnp.log(segsum)
    lse_b = jnp.dot(lse, e8t, preferred_element_type=jnp.float32,
                    precision=lax.Precision.HIGHEST)
    out_ref[...] = h - m - lse_b


_tc_c = pl.pallas_call(
    _tc_c_body,
    grid=(NP // 8 // _RI,),
    in_specs=[
        pl.BlockSpec((NC, _RI, 128), lambda i: (0, i, 0)),
        pl.BlockSpec((NC, _RI, 128), lambda i: (0, i, 0)),
        pl.BlockSpec((128,), lambda i: (0,)),
    ],
    out_specs=pl.BlockSpec((_RI, 128), lambda i: (i, 0)),
    out_shape=jax.ShapeDtypeStruct((NP // 8, 128), jnp.float32),
)


def kernel(x, edge_index, W1, b1, W2, b2):
    npad = E_PAD - E
    src_pad = jnp.arange(npad, dtype=jnp.int32) % N
    dst_pad = N + jnp.arange(npad, dtype=jnp.int32) % (NP - N)
    src = jnp.concatenate(
        [edge_index[0].astype(jnp.int32), src_pad]).reshape(NW, NCHUNK, CHUNK)
    dst = jnp.concatenate(
        [edge_index[1].astype(jnp.int32), dst_pad]).reshape(NW, NCHUNK, CHUNK)

    W1p = jnp.pad(W1, ((0, 0), (0, H1P - H1)))
    b1p = jnp.pad(b1, (0, H1P - H1))
    W2p = jnp.pad(W2, ((0, H1P - H1), (0, 0)))

    degp = _deg_kernel(dst)
    hs1, dinv = _tc_a(x, W1p, degp)
    aggp1 = _agg64(hs1, src, dst)
    hs2 = _tc_b(aggp1, dinv, W2p, b1p)
    aggp2 = _agg16(hs2, src, dst)
    out128 = _tc_c(aggp2.reshape(NC, NP // 8, 128),
                   degp.reshape(NC, NP // 8, 128),
                   jnp.tile(b2, 8))
    return out128.reshape(NP, H2)[:N]
